# baseline (speedup 1.0000x reference)
import functools
import math

import jax
import jax.numpy as jnp
from jax import lax
from jax.experimental import pallas as pl
from jax.experimental.pallas import tpu as pltpu
from jax.experimental.pallas import tpu_sc as plsc

_C1 = 2146121005
_C2 = 2221713035
_MERSENNE = (2**17 - 1, 2**19 - 1, 2**31 - 1)
_LANES = 128
_NC, _NS = 2, 16
_NW = _NC * _NS
_BR = 32
_BW = 8
_SUB = 2048
_NCHUNK = 2


def _u(v):
    return jnp.uint32(v)


def _asr(x_u32, n):
    xi = lax.bitcast_convert_type(x_u32, jnp.int32)
    return lax.bitcast_convert_type(
        lax.shift_right_arithmetic(xi, jnp.int32(n)), jnp.uint32)


def _wide_mul_const(a, c):
    m16 = _u(0xFFFF)
    a0 = a & m16
    a1 = a >> _u(16)
    b0 = _u(c & 0xFFFF)
    b1 = _u((c >> 16) & 0xFFFF)
    w0 = a0 * b0
    t = a1 * b0 + (w0 >> _u(16))
    t2 = a0 * b1 + (t & m16)
    lo = (t2 << _u(16)) | (w0 & m16)
    hi = a1 * b1 + (t >> _u(16)) + (t2 >> _u(16))
    return hi, lo


def _xs(hi, lo, n):
    s_hi = _asr(hi, n)
    s_lo = (hi << _u(32 - n)) | (lo >> _u(n))
    return hi ^ s_hi, lo ^ s_lo


def _mc(hi, lo, c):
    ph, plo = _wide_mul_const(lo, c)
    return hi * _u(c) + ph, plo


def _mod_size(hi, lo, size):
    r32 = (1 << 32) % size
    s64 = (1 << 64) % size
    magic = (1 << 55) // size
    h, l = hi, lo
    for _ in range(3):
        ph, plo = _wide_mul_const(h, r32)
        l2 = plo + l
        carry = jnp.where(l2 < plo, _u(1), _u(0))
        h, l = ph + carry, l2
    s = h * _u(r32)
    v = s + l
    add1 = jnp.where(v < s, _u(r32), _u(0))
    v1 = v + add1
    add2 = jnp.where(v1 < add1, _u(r32), _u(0))
    v = v1 + add2
    qh, _ = _wide_mul_const(v, magic)
    q = qh >> _u(23)
    r = v - q * _u(size)
    r = jnp.where(r >= _u(size), r - _u(size), r)
    neg = lax.bitcast_convert_type(hi, jnp.int32) < jnp.int32(0)
    r_neg = r + jnp.where(r < _u(s64), _u(size), _u(0)) - _u(s64)
    r = jnp.where(neg, r_neg, r)
    return lax.bitcast_convert_type(r, jnp.int32)


def _pack_body(b_ref, w_ref):
    acc = b_ref[0].astype(jnp.uint32)
    for p in range(1, 32):
        acc = acc | (b_ref[p].astype(jnp.uint32) << _u(p))
    w_ref[...] = lax.bitcast_convert_type(acc, jnp.int32)


def _hash_body(t_ref, dep_ref, idx_ref, *, rounds, size):
    del dep_ref
    t0 = t_ref[0].astype(jnp.uint32)
    t1 = t_ref[1].astype(jnp.uint32)
    t2 = t_ref[2].astype(jnp.uint32)
    hi = jnp.zeros(t0.shape, jnp.uint32)
    lo = jnp.zeros(t0.shape, jnp.uint32)
    for m, tk in zip(_MERSENNE, (t0, t1, t2)):
        ph, plo = _wide_mul_const(tk, m)
        l2 = lo + plo
        carry = jnp.where(l2 < plo, _u(1), _u(0))
        hi, lo = hi + ph + carry, l2
    for r in range(rounds):
        hi, lo = _xs(hi, lo, 16)
        hi, lo = _mc(hi, lo, _C1)
        hi, lo = _xs(hi, lo, 15)
        hi, lo = _mc(hi, lo, _C2)
        hi, lo = _xs(hi, lo, 16)
        idx_ref[r] = _mod_size(hi, lo, size)


def _gather_body(idx_hbm, table_hbm, out_hbm, *refs, rounds, chunk, sub):
    idx_vs = refs[:rounds]
    widx_vs = refs[rounds:2 * rounds]
    got_vs = refs[2 * rounds:3 * rounds]
    out_v = refs[3 * rounds]
    sem = refs[3 * rounds + 1]
    spm = refs[3 * rounds + 2]

    sid = lax.axis_index("s")

    @pl.when(sid == jnp.int32(0))
    def _stage():
        pltpu.sync_copy(table_hbm, spm)

    plsc.subcore_barrier()

    wid = sid * jnp.int32(_NC) + lax.axis_index("c")
    base0 = wid * jnp.int32(chunk)

    def body(i, _):
        base = base0 + i * jnp.int32(sub)
        for r in range(rounds):
            pltpu.sync_copy(idx_hbm.at[jnp.int32(r), pl.ds(base, sub)],
                            idx_vs[r])

        def widx_body(j, _):
            o = j * jnp.int32(16)
            for r in range(rounds):
                widx_vs[r][pl.ds(o, 16)] = lax.shift_right_logical(
                    idx_vs[r][pl.ds(o, 16)], jnp.int32(5))
            return 0

        lax.fori_loop(jnp.int32(0), jnp.int32(sub // 16), widx_body, 0)

        cps = [pltpu.async_copy(spm.at[widx_vs[r]], got_vs[r], sem)
               for r in range(rounds)]
        for c in cps:
            c.wait()

        def and_body(j, _):
            o = j * jnp.int32(16)
            acc = None
            for r in range(rounds):
                sh = idx_vs[r][pl.ds(o, 16)] & jnp.int32(31)
                bit = lax.shift_right_logical(got_vs[r][pl.ds(o, 16)], sh)
                bit = bit & jnp.int32(1)
                acc = bit if acc is None else (acc & bit)
            out_v[pl.ds(o, 16)] = acc ^ jnp.int32(1)
            return 0

        lax.fori_loop(jnp.int32(0), jnp.int32(sub // 16), and_body, 0)
        pltpu.sync_copy(out_v, out_hbm.at[pl.ds(base, sub)])
        return 0

    lax.fori_loop(jnp.int32(0), jnp.int32(chunk // sub), body, 0)


def kernel(negative_batch, bit_array, mersenne, rounds):
    batch, num_neg, _ = negative_batch.shape
    b = batch * num_neg
    size = bit_array.shape[0]
    try:
        r_static = int(rounds)
    except Exception:
        r_static = int(math.ceil(size / 1_000_000 * math.log(2)))

    nrow = b // _LANES
    t3 = (negative_batch.astype(jnp.int32)
          .reshape(b, 3).transpose(1, 0).reshape(3, nrow, _LANES))

    blk = _BW * _LANES
    w_pad = -(-(size // 32 + 1) // blk) * blk
    size_pad = 32 * w_pad
    bits8 = (jnp.pad(bit_array, (0, size_pad - size)).astype(jnp.int8)
             .reshape(w_pad, 32).transpose(1, 0)
             .reshape(32, w_pad // _LANES, _LANES))
    packed = pl.pallas_call(
        _pack_body,
        grid=(w_pad // blk,),
        in_specs=[pl.BlockSpec(
            (32, _BW, _LANES),
            lambda i: (jnp.int32(0), i, jnp.int32(0)))],
        out_specs=pl.BlockSpec(
            (_BW, _LANES), lambda i: (i, jnp.int32(0))),
        out_shape=jax.ShapeDtypeStruct((w_pad // _LANES, _LANES), jnp.int32),
    )(bits8)
    table = packed.reshape(w_pad)

    bc = b // _NCHUNK
    nrow_c = nrow // _NCHUNK
    chunk = bc // _NW

    mesh = plsc.VectorSubcoreMesh(
        core_axis_name="c", subcore_axis_name="s",
        num_cores=_NC, num_subcores=_NS)
    sc_gather = pl.kernel(
        functools.partial(_gather_body, rounds=r_static, chunk=chunk,
                          sub=_SUB),
        out_type=jax.ShapeDtypeStruct((bc,), jnp.int32),
        mesh=mesh,
        scratch_types=(
            [pltpu.VMEM((_SUB,), jnp.int32) for _ in range(3 * r_static)]
            + [pltpu.VMEM((_SUB,), jnp.int32), pltpu.SemaphoreType.DMA,
               pltpu.VMEM_SHARED((w_pad,), jnp.int32)]
        ),
    )

    outs = []
    for c in range(_NCHUNK):
        tc = lax.slice_in_dim(t3, c * nrow_c, (c + 1) * nrow_c, axis=1)
        idx = pl.pallas_call(
            functools.partial(_hash_body, rounds=r_static, size=size),
            grid=(nrow_c // _BR,),
            in_specs=[
                pl.BlockSpec(
                    (3, _BR, _LANES),
                    lambda i: (jnp.int32(0), i, jnp.int32(0))),
                pl.BlockSpec(
                    (_BW, _LANES),
                    lambda i: (jnp.int32(0), jnp.int32(0))),
            ],
            out_specs=pl.BlockSpec(
                (r_static, _BR, _LANES),
                lambda i: (jnp.int32(0), i, jnp.int32(0))),
            out_shape=jax.ShapeDtypeStruct(
                (r_static, nrow_c, _LANES), jnp.int32),
        )(tc, packed)
        outs.append(sc_gather(idx.reshape(r_static, bc), table))

    out = jnp.concatenate(outs)
    return out.reshape(batch, num_neg).astype(bool)


# speedup vs baseline: 1.2770x; 1.2770x over previous
import functools
import math

import jax
import jax.numpy as jnp
from jax import lax
from jax.experimental import pallas as pl
from jax.experimental.pallas import tpu as pltpu
from jax.experimental.pallas import tpu_sc as plsc

_C1 = 2146121005
_C2 = 2221713035
_MERSENNE = (2**17 - 1, 2**19 - 1, 2**31 - 1)
_LANES = 128
_NC, _NS = 2, 16
_NW = _NC * _NS
_BR = 32
_BW = 8
_SUB = 2048
_SR = _SUB // _LANES
_NCHUNK = 2


def _u(v):
    return jnp.uint32(v)


def _asr(x_u32, n):
    xi = lax.bitcast_convert_type(x_u32, jnp.int32)
    return lax.bitcast_convert_type(
        lax.shift_right_arithmetic(xi, jnp.int32(n)), jnp.uint32)


def _wide_mul_const(a, c):
    m16 = _u(0xFFFF)
    a0 = a & m16
    a1 = a >> _u(16)
    b0 = _u(c & 0xFFFF)
    b1 = _u((c >> 16) & 0xFFFF)
    w0 = a0 * b0
    t = a1 * b0 + (w0 >> _u(16))
    t2 = a0 * b1 + (t & m16)
    lo = (t2 << _u(16)) | (w0 & m16)
    hi = a1 * b1 + (t >> _u(16)) + (t2 >> _u(16))
    return hi, lo


def _xs(hi, lo, n):
    s_hi = _asr(hi, n)
    s_lo = (hi << _u(32 - n)) | (lo >> _u(n))
    return hi ^ s_hi, lo ^ s_lo


def _mc(hi, lo, c):
    ph, plo = _wide_mul_const(lo, c)
    return hi * _u(c) + ph, plo


def _mod_size(hi, lo, size):
    r32 = (1 << 32) % size
    s64 = (1 << 64) % size
    magic = (1 << 55) // size
    h, l = hi, lo
    for _ in range(3):
        ph, plo = _wide_mul_const(h, r32)
        l2 = plo + l
        carry = jnp.where(l2 < plo, _u(1), _u(0))
        h, l = ph + carry, l2
    s = h * _u(r32)
    v = s + l
    add1 = jnp.where(v < s, _u(r32), _u(0))
    v1 = v + add1
    add2 = jnp.where(v1 < add1, _u(r32), _u(0))
    v = v1 + add2
    qh, _ = _wide_mul_const(v, magic)
    q = qh >> _u(23)
    r = v - q * _u(size)
    r = jnp.where(r >= _u(size), r - _u(size), r)
    neg = lax.bitcast_convert_type(hi, jnp.int32) < jnp.int32(0)
    r_neg = r + jnp.where(r < _u(s64), _u(size), _u(0)) - _u(s64)
    r = jnp.where(neg, r_neg, r)
    return lax.bitcast_convert_type(r, jnp.int32)


def _pack_body(b_ref, w_ref):
    acc = b_ref[0].astype(jnp.uint32)
    for p in range(1, 32):
        acc = acc | (b_ref[p].astype(jnp.uint32) << _u(p))
    w_ref[...] = lax.bitcast_convert_type(acc, jnp.int32)


def _hash_body(t_ref, dep_ref, idx_ref, *, rounds, size):
    del dep_ref
    t0 = t_ref[0].astype(jnp.uint32)
    t1 = t_ref[1].astype(jnp.uint32)
    t2 = t_ref[2].astype(jnp.uint32)
    hi = jnp.zeros(t0.shape, jnp.uint32)
    lo = jnp.zeros(t0.shape, jnp.uint32)
    for m, tk in zip(_MERSENNE, (t0, t1, t2)):
        ph, plo = _wide_mul_const(tk, m)
        l2 = lo + plo
        carry = jnp.where(l2 < plo, _u(1), _u(0))
        hi, lo = hi + ph + carry, l2
    nq = _BR // _SR
    for r in range(rounds):
        hi, lo = _xs(hi, lo, 16)
        hi, lo = _mc(hi, lo, _C1)
        hi, lo = _xs(hi, lo, 15)
        hi, lo = _mc(hi, lo, _C2)
        hi, lo = _xs(hi, lo, 16)
        val = _mod_size(hi, lo, size)
        for q in range(nq):
            idx_ref[q, r] = val[q * _SR:(q + 1) * _SR]


def _gather_body(idx_hbm, table_hbm, out_hbm, *refs, rounds, chunk, sub):
    widx_vs = refs[:rounds]
    got_vs = refs[rounds:2 * rounds]
    buf0 = refs[2 * rounds]
    buf1 = refs[2 * rounds + 1]
    out_v = refs[2 * rounds + 2]
    sem_idx = refs[2 * rounds + 3]
    sem_g = refs[2 * rounds + 4]
    spm = refs[2 * rounds + 5]

    sid = lax.axis_index("s")

    @pl.when(sid == jnp.int32(0))
    def _stage():
        pltpu.sync_copy(table_hbm, spm)

    plsc.subcore_barrier()

    wid = sid * jnp.int32(_NC) + lax.axis_index("c")
    nsub = chunk // sub
    sub0 = wid * jnp.int32(nsub)
    ng = _LANES // 16

    pltpu.async_copy(idx_hbm.at[sub0], buf0, sem_idx)

    def outer(i2, _):
        for par, buf in ((0, buf0), (1, buf1)):
            i = i2 * jnp.int32(2) + jnp.int32(par)
            s = sub0 + i
            pltpu.make_async_copy(idx_hbm.at[s], buf, sem_idx).wait()

            @pl.when(i < jnp.int32(nsub - 1))
            def _prefetch():
                nxt = buf1 if par == 0 else buf0
                pltpu.async_copy(idx_hbm.at[s + jnp.int32(1)], nxt, sem_idx)

            def widx_body(row, _):
                ro = row * jnp.int32(_LANES)
                for r in range(rounds):
                    for g in range(ng):
                        v = buf[jnp.int32(r), row,
                                pl.ds(jnp.int32(g * 16), 16)]
                        widx_vs[r][pl.ds(ro + jnp.int32(g * 16), 16)] = (
                            lax.shift_right_logical(v, jnp.int32(5)))
                return 0

            lax.fori_loop(jnp.int32(0), jnp.int32(_SR), widx_body, 0)

            cps = [pltpu.async_copy(spm.at[widx_vs[r]], got_vs[r], sem_g)
                   for r in range(rounds)]
            for c in cps:
                c.wait()

            def and_body(row, _):
                ro = row * jnp.int32(_LANES)
                for g in range(ng):
                    o = ro + jnp.int32(g * 16)
                    acc = None
                    for r in range(rounds):
                        sh = buf[jnp.int32(r), row,
                                 pl.ds(jnp.int32(g * 16), 16)] & jnp.int32(31)
                        t = lax.shift_right_logical(got_vs[r][pl.ds(o, 16)],
                                                    sh)
                        acc = t if acc is None else acc & t
                    out_v[pl.ds(o, 16)] = (acc & jnp.int32(1)) ^ jnp.int32(1)
                return 0

            lax.fori_loop(jnp.int32(0), jnp.int32(_SR), and_body, 0)
            pltpu.sync_copy(out_v, out_hbm.at[pl.ds(s * jnp.int32(sub), sub)])
        return 0

    lax.fori_loop(jnp.int32(0), jnp.int32(nsub // 2), outer, 0)


def kernel(negative_batch, bit_array, mersenne, rounds):
    batch, num_neg, _ = negative_batch.shape
    b = batch * num_neg
    size = bit_array.shape[0]
    try:
        r_static = int(rounds)
    except Exception:
        r_static = int(math.ceil(size / 1_000_000 * math.log(2)))

    nrow = b // _LANES
    t3 = (negative_batch.astype(jnp.int32)
          .reshape(b, 3).transpose(1, 0).reshape(3, nrow, _LANES))

    blk = _BW * _LANES
    w_pad = -(-(size // 32 + 1) // blk) * blk
    size_pad = 32 * w_pad
    bits8 = (jnp.pad(bit_array, (0, size_pad - size)).astype(jnp.int8)
             .reshape(w_pad, 32).transpose(1, 0)
             .reshape(32, w_pad // _LANES, _LANES))
    packed = pl.pallas_call(
        _pack_body,
        grid=(w_pad // blk,),
        in_specs=[pl.BlockSpec(
            (32, _BW, _LANES),
            lambda i: (jnp.int32(0), i, jnp.int32(0)))],
        out_specs=pl.BlockSpec(
            (_BW, _LANES), lambda i: (i, jnp.int32(0))),
        out_shape=jax.ShapeDtypeStruct((w_pad // _LANES, _LANES), jnp.int32),
    )(bits8)
    table = packed.reshape(w_pad)

    bc = b // _NCHUNK
    nrow_c = nrow // _NCHUNK
    chunk = bc // _NW

    mesh = plsc.VectorSubcoreMesh(
        core_axis_name="c", subcore_axis_name="s",
        num_cores=_NC, num_subcores=_NS)
    sc_gather = pl.kernel(
        functools.partial(_gather_body, rounds=r_static, chunk=chunk,
                          sub=_SUB),
        out_type=jax.ShapeDtypeStruct((bc,), jnp.int32),
        mesh=mesh,
        scratch_types=(
            [pltpu.VMEM((_SUB,), jnp.int32) for _ in range(2 * r_static)]
            + [pltpu.VMEM((r_static, _SR, _LANES), jnp.int32),
               pltpu.VMEM((r_static, _SR, _LANES), jnp.int32),
               pltpu.VMEM((_SUB,), jnp.int32),
               pltpu.SemaphoreType.DMA, pltpu.SemaphoreType.DMA,
               pltpu.VMEM_SHARED((w_pad,), jnp.int32)]
        ),
    )

    outs = []
    for c in range(_NCHUNK):
        tc = lax.slice_in_dim(t3, c * nrow_c, (c + 1) * nrow_c, axis=1)
        idx = pl.pallas_call(
            functools.partial(_hash_body, rounds=r_static, size=size),
            grid=(nrow_c // _BR,),
            in_specs=[
                pl.BlockSpec(
                    (3, _BR, _LANES),
                    lambda i: (jnp.int32(0), i, jnp.int32(0))),
                pl.BlockSpec(
                    (_BW, _LANES),
                    lambda i: (jnp.int32(0), jnp.int32(0))),
            ],
            out_specs=pl.BlockSpec(
                (_BR // _SR, r_static, _SR, _LANES),
                lambda i: (i, jnp.int32(0), jnp.int32(0), jnp.int32(0))),
            out_shape=jax.ShapeDtypeStruct(
                (nrow_c // _SR, r_static, _SR, _LANES), jnp.int32),
        )(tc, packed)
        outs.append(sc_gather(idx, table))

    out = jnp.concatenate(outs)
    return out.reshape(batch, num_neg).astype(bool)


# speedup vs baseline: 1.9703x; 1.5428x over previous
import functools
import math

import jax
import jax.numpy as jnp
from jax import lax
from jax.experimental import pallas as pl
from jax.experimental.pallas import tpu as pltpu
from jax.experimental.pallas import tpu_sc as plsc

_C1 = 2146121005
_C2 = 2221713035
_MERSENNE = (2**17 - 1, 2**19 - 1, 2**31 - 1)
_LANES = 128
_NC, _NS = 2, 16
_NW = _NC * _NS
_BR = 32
_BW = 64
_SUB = 2048
_SR = _SUB // _LANES
_NCHUNK = 2


def _u(v):
    return jnp.uint32(v)


def _asr(x_u32, n):
    xi = lax.bitcast_convert_type(x_u32, jnp.int32)
    return lax.bitcast_convert_type(
        lax.shift_right_arithmetic(xi, jnp.int32(n)), jnp.uint32)


def _wide_mul_const(a, c):
    m16 = _u(0xFFFF)
    a0 = a & m16
    a1 = a >> _u(16)
    b0 = _u(c & 0xFFFF)
    b1 = _u((c >> 16) & 0xFFFF)
    w0 = a0 * b0
    t = a1 * b0 + (w0 >> _u(16))
    t2 = a0 * b1 + (t & m16)
    lo = (t2 << _u(16)) | (w0 & m16)
    hi = a1 * b1 + (t >> _u(16)) + (t2 >> _u(16))
    return hi, lo


def _xs(hi, lo, n):
    s_hi = _asr(hi, n)
    s_lo = (hi << _u(32 - n)) | (lo >> _u(n))
    return hi ^ s_hi, lo ^ s_lo


def _mc(hi, lo, c):
    ph, plo = _wide_mul_const(lo, c)
    return hi * _u(c) + ph, plo


def _mod_size(hi, lo, size):
    r32 = (1 << 32) % size
    s64 = (1 << 64) % size
    magic = (1 << 55) // size
    h, l = hi, lo
    for _ in range(3):
        ph, plo = _wide_mul_const(h, r32)
        l2 = plo + l
        carry = jnp.where(l2 < plo, _u(1), _u(0))
        h, l = ph + carry, l2
    s = h * _u(r32)
    v = s + l
    add1 = jnp.where(v < s, _u(r32), _u(0))
    v1 = v + add1
    add2 = jnp.where(v1 < add1, _u(r32), _u(0))
    v = v1 + add2
    qh, _ = _wide_mul_const(v, magic)
    q = qh >> _u(23)
    r = v - q * _u(size)
    r = jnp.where(r >= _u(size), r - _u(size), r)
    neg = lax.bitcast_convert_type(hi, jnp.int32) < jnp.int32(0)
    r_neg = r + jnp.where(r < _u(s64), _u(size), _u(0)) - _u(s64)
    r = jnp.where(neg, r_neg, r)
    return lax.bitcast_convert_type(r, jnp.int32)


def _pack_body(b_ref, w0, w1, w2, w3, out_ref):
    x = b_ref[...]
    dn = (((1,), (0,)), ((), ()))
    m = jnp.int32(0xFF)
    accs = []
    for w in (w0, w1, w2, w3):
        accs.append(lax.dot_general(
            x, w[...], dn, preferred_element_type=jnp.int32) & m)
    out_ref[...] = (accs[0] | (accs[1] << jnp.int32(8))
                    | (accs[2] << jnp.int32(16))
                    | (accs[3] << jnp.int32(24)))


def _pack_weights():
    import numpy as np
    w = np.zeros((4, 4096, 128), np.int16)
    for l in range(128):
        q, rem = divmod(l, 4)
        for h in range(4):
            for k in range(8):
                lane = 32 * rem + 8 * h + k
                w[h, 128 * q + lane, l] = 1 << k
    return jnp.asarray(w.astype(np.int8))


def _hash_body(t_ref, dep_ref, idx_ref, *, rounds, size):
    del dep_ref
    t0 = t_ref[0].astype(jnp.uint32)
    t1 = t_ref[1].astype(jnp.uint32)
    t2 = t_ref[2].astype(jnp.uint32)
    hi = jnp.zeros(t0.shape, jnp.uint32)
    lo = jnp.zeros(t0.shape, jnp.uint32)
    for m, tk in zip(_MERSENNE, (t0, t1, t2)):
        ph, plo = _wide_mul_const(tk, m)
        l2 = lo + plo
        carry = jnp.where(l2 < plo, _u(1), _u(0))
        hi, lo = hi + ph + carry, l2
    nq = _BR // _SR
    for r in range(rounds):
        hi, lo = _xs(hi, lo, 16)
        hi, lo = _mc(hi, lo, _C1)
        hi, lo = _xs(hi, lo, 15)
        hi, lo = _mc(hi, lo, _C2)
        hi, lo = _xs(hi, lo, 16)
        val = _mod_size(hi, lo, size)
        for q in range(nq):
            idx_ref[q, r] = val[q * _SR:(q + 1) * _SR]


def _gather_body(idx_hbm, table_hbm, out_hbm, *refs, rounds, chunk, sub):
    widx_vs = refs[:rounds]
    got_vs = refs[rounds:2 * rounds]
    buf0 = refs[2 * rounds]
    buf1 = refs[2 * rounds + 1]
    out_v = refs[2 * rounds + 2]
    sem_idx = refs[2 * rounds + 3]
    sem_g = refs[2 * rounds + 4]
    spm = refs[2 * rounds + 5]

    sid = lax.axis_index("s")

    @pl.when(sid == jnp.int32(0))
    def _stage():
        pltpu.sync_copy(table_hbm, spm)

    plsc.subcore_barrier()

    wid = sid * jnp.int32(_NC) + lax.axis_index("c")
    nsub = chunk // sub
    sub0 = wid * jnp.int32(nsub)
    ng = _LANES // 16

    pltpu.async_copy(idx_hbm.at[sub0], buf0, sem_idx)

    def outer(i2, _):
        for par, buf in ((0, buf0), (1, buf1)):
            i = i2 * jnp.int32(2) + jnp.int32(par)
            s = sub0 + i
            pltpu.make_async_copy(idx_hbm.at[s], buf, sem_idx).wait()

            @pl.when(i < jnp.int32(nsub - 1))
            def _prefetch():
                nxt = buf1 if par == 0 else buf0
                pltpu.async_copy(idx_hbm.at[s + jnp.int32(1)], nxt, sem_idx)

            def widx_body(row, _):
                ro = row * jnp.int32(_LANES)
                for r in range(rounds):
                    for g in range(ng):
                        v = buf[jnp.int32(r), row,
                                pl.ds(jnp.int32(g * 16), 16)]
                        widx_vs[r][pl.ds(ro + jnp.int32(g * 16), 16)] = (
                            lax.shift_right_logical(v, jnp.int32(5)))
                return 0

            lax.fori_loop(jnp.int32(0), jnp.int32(_SR), widx_body, 0)

            cps = [pltpu.async_copy(spm.at[widx_vs[r]], got_vs[r], sem_g)
                   for r in range(rounds)]
            for c in cps:
                c.wait()

            def and_body(row, _):
                ro = row * jnp.int32(_LANES)
                for g in range(ng):
                    o = ro + jnp.int32(g * 16)
                    acc = None
                    for r in range(rounds):
                        sh = buf[jnp.int32(r), row,
                                 pl.ds(jnp.int32(g * 16), 16)] & jnp.int32(31)
                        t = lax.shift_right_logical(got_vs[r][pl.ds(o, 16)],
                                                    sh)
                        acc = t if acc is None else acc & t
                    out_v[pl.ds(o, 16)] = (acc & jnp.int32(1)) ^ jnp.int32(1)
                return 0

            lax.fori_loop(jnp.int32(0), jnp.int32(_SR), and_body, 0)
            pltpu.sync_copy(out_v, out_hbm.at[pl.ds(s * jnp.int32(sub), sub)])
        return 0

    lax.fori_loop(jnp.int32(0), jnp.int32(nsub // 2), outer, 0)


def kernel(negative_batch, bit_array, mersenne, rounds):
    batch, num_neg, _ = negative_batch.shape
    b = batch * num_neg
    size = bit_array.shape[0]
    try:
        r_static = int(rounds)
    except Exception:
        r_static = int(math.ceil(size / 1_000_000 * math.log(2)))

    nrow = b // _LANES
    t3 = (negative_batch.astype(jnp.int32)
          .reshape(b, 3).transpose(1, 0).reshape(3, nrow, _LANES))

    blk = _BW * _LANES
    w_pad = -(-(size // 32 + 1) // blk) * blk
    size_pad = 32 * w_pad
    bits8 = (jnp.pad(bit_array, (0, size_pad - size)).astype(jnp.int8)
             .reshape(size_pad // 4096, 4096))
    wts = _pack_weights()
    wspec = pl.BlockSpec(
        (4096, _LANES), lambda i: (jnp.int32(0), jnp.int32(0)))
    packed = pl.pallas_call(
        _pack_body,
        grid=(size_pad // 4096 // _BW,),
        in_specs=[pl.BlockSpec(
            (_BW, 4096), lambda i: (i, jnp.int32(0))),
            wspec, wspec, wspec, wspec],
        out_specs=pl.BlockSpec(
            (_BW, _LANES), lambda i: (i, jnp.int32(0))),
        out_shape=jax.ShapeDtypeStruct((w_pad // _LANES, _LANES), jnp.int32),
    )(bits8, wts[0], wts[1], wts[2], wts[3])
    table = packed.reshape(w_pad)

    bc = b // _NCHUNK
    nrow_c = nrow // _NCHUNK
    chunk = bc // _NW

    mesh = plsc.VectorSubcoreMesh(
        core_axis_name="c", subcore_axis_name="s",
        num_cores=_NC, num_subcores=_NS)
    sc_gather = pl.kernel(
        functools.partial(_gather_body, rounds=r_static, chunk=chunk,
                          sub=_SUB),
        out_type=jax.ShapeDtypeStruct((bc,), jnp.int32),
        mesh=mesh,
        scratch_types=(
            [pltpu.VMEM((_SUB,), jnp.int32) for _ in range(2 * r_static)]
            + [pltpu.VMEM((r_static, _SR, _LANES), jnp.int32),
               pltpu.VMEM((r_static, _SR, _LANES), jnp.int32),
               pltpu.VMEM((_SUB,), jnp.int32),
               pltpu.SemaphoreType.DMA, pltpu.SemaphoreType.DMA,
               pltpu.VMEM_SHARED((w_pad,), jnp.int32)]
        ),
    )

    outs = []
    for c in range(_NCHUNK):
        tc = lax.slice_in_dim(t3, c * nrow_c, (c + 1) * nrow_c, axis=1)
        idx = pl.pallas_call(
            functools.partial(_hash_body, rounds=r_static, size=size),
            grid=(nrow_c // _BR,),
            in_specs=[
                pl.BlockSpec(
                    (3, _BR, _LANES),
                    lambda i: (jnp.int32(0), i, jnp.int32(0))),
                pl.BlockSpec(
                    (_BW, _LANES),
                    lambda i: (jnp.int32(0), jnp.int32(0))),
            ],
            out_specs=pl.BlockSpec(
                (_BR // _SR, r_static, _SR, _LANES),
                lambda i: (i, jnp.int32(0), jnp.int32(0), jnp.int32(0))),
            out_shape=jax.ShapeDtypeStruct(
                (nrow_c // _SR, r_static, _SR, _LANES), jnp.int32),
        )(tc, packed)
        outs.append(sc_gather(idx, table))

    out = jnp.concatenate(outs)
    return out.reshape(batch, num_neg).astype(bool)


# speedup vs baseline: 2.0745x; 1.0529x over previous
import functools
import math

import jax
import jax.numpy as jnp
from jax import lax
from jax.experimental import pallas as pl
from jax.experimental.pallas import tpu as pltpu
from jax.experimental.pallas import tpu_sc as plsc

_C1 = 2146121005
_C2 = 2221713035
_MERSENNE = (2**17 - 1, 2**19 - 1, 2**31 - 1)
_LANES = 128
_NC, _NS = 2, 16
_NW = _NC * _NS
_BR = 32
_BW = 64
_SUB = 2048
_SR = _SUB // _LANES
_NCHUNK = 4


def _u(v):
    return jnp.uint32(v)


def _asr(x_u32, n):
    xi = lax.bitcast_convert_type(x_u32, jnp.int32)
    return lax.bitcast_convert_type(
        lax.shift_right_arithmetic(xi, jnp.int32(n)), jnp.uint32)


def _wide_mul_const(a, c):
    m16 = _u(0xFFFF)
    a0 = a & m16
    a1 = a >> _u(16)
    b0 = _u(c & 0xFFFF)
    b1 = _u((c >> 16) & 0xFFFF)
    w0 = a0 * b0
    t = a1 * b0 + (w0 >> _u(16))
    t2 = a0 * b1 + (t & m16)
    lo = (t2 << _u(16)) | (w0 & m16)
    hi = a1 * b1 + (t >> _u(16)) + (t2 >> _u(16))
    return hi, lo


def _xs(hi, lo, n):
    s_hi = _asr(hi, n)
    s_lo = (hi << _u(32 - n)) | (lo >> _u(n))
    return hi ^ s_hi, lo ^ s_lo


def _mc(hi, lo, c):
    ph, plo = _wide_mul_const(lo, c)
    return hi * _u(c) + ph, plo


def _mod_size(hi, lo, size):
    r32 = (1 << 32) % size
    s64 = (1 << 64) % size
    magic = (1 << 55) // size
    h, l = hi, lo
    for _ in range(3):
        ph, plo = _wide_mul_const(h, r32)
        l2 = plo + l
        carry = jnp.where(l2 < plo, _u(1), _u(0))
        h, l = ph + carry, l2
    s = h * _u(r32)
    v = s + l
    add1 = jnp.where(v < s, _u(r32), _u(0))
    v1 = v + add1
    add2 = jnp.where(v1 < add1, _u(r32), _u(0))
    v = v1 + add2
    qh, _ = _wide_mul_const(v, magic)
    q = qh >> _u(23)
    r = v - q * _u(size)
    r = jnp.where(r >= _u(size), r - _u(size), r)
    neg = lax.bitcast_convert_type(hi, jnp.int32) < jnp.int32(0)
    r_neg = r + jnp.where(r < _u(s64), _u(size), _u(0)) - _u(s64)
    r = jnp.where(neg, r_neg, r)
    return lax.bitcast_convert_type(r, jnp.int32)


def _pack_body(b_ref, w0, w1, w2, w3, out_ref):
    x = b_ref[...]
    dn = (((1,), (0,)), ((), ()))
    m = jnp.int32(0xFF)
    accs = []
    for w in (w0, w1, w2, w3):
        accs.append(lax.dot_general(
            x, w[...], dn, preferred_element_type=jnp.int32) & m)
    out_ref[...] = (accs[0] | (accs[1] << jnp.int32(8))
                    | (accs[2] << jnp.int32(16))
                    | (accs[3] << jnp.int32(24)))


def _pack_weights():
    import numpy as np
    w = np.zeros((4, 4096, 128), np.int16)
    for l in range(128):
        q, rem = divmod(l, 4)
        for h in range(4):
            for k in range(8):
                lane = 32 * rem + 8 * h + k
                w[h, 128 * q + lane, l] = 1 << k
    return jnp.asarray(w.astype(np.int8))


def _hash_body(t_ref, dep_ref, idx_ref, *, rounds, size):
    del dep_ref
    t0 = t_ref[0].astype(jnp.uint32)
    t1 = t_ref[1].astype(jnp.uint32)
    t2 = t_ref[2].astype(jnp.uint32)
    hi = jnp.zeros(t0.shape, jnp.uint32)
    lo = jnp.zeros(t0.shape, jnp.uint32)
    for m, tk in zip(_MERSENNE, (t0, t1, t2)):
        ph, plo = _wide_mul_const(tk, m)
        l2 = lo + plo
        carry = jnp.where(l2 < plo, _u(1), _u(0))
        hi, lo = hi + ph + carry, l2
    nq = _BR // _SR
    for r in range(rounds):
        hi, lo = _xs(hi, lo, 16)
        hi, lo = _mc(hi, lo, _C1)
        hi, lo = _xs(hi, lo, 15)
        hi, lo = _mc(hi, lo, _C2)
        hi, lo = _xs(hi, lo, 16)
        val = _mod_size(hi, lo, size)
        for q in range(nq):
            idx_ref[q, r] = val[q * _SR:(q + 1) * _SR]


def _gather_body(idx_hbm, table_hbm, out_hbm, *refs, rounds, chunk, sub):
    widx_vs = refs[:rounds]
    got_vs = refs[rounds:2 * rounds]
    buf0 = refs[2 * rounds]
    buf1 = refs[2 * rounds + 1]
    out_v = refs[2 * rounds + 2]
    sem_idx = refs[2 * rounds + 3]
    sem_g = refs[2 * rounds + 4]
    spm = refs[2 * rounds + 5]

    sid = lax.axis_index("s")

    @pl.when(sid == jnp.int32(0))
    def _stage():
        pltpu.sync_copy(table_hbm, spm)

    plsc.subcore_barrier()

    wid = sid * jnp.int32(_NC) + lax.axis_index("c")
    nsub = chunk // sub
    sub0 = wid * jnp.int32(nsub)
    ng = _LANES // 16

    pltpu.async_copy(idx_hbm.at[sub0], buf0, sem_idx)

    def outer(i2, _):
        for par, buf in ((0, buf0), (1, buf1)):
            i = i2 * jnp.int32(2) + jnp.int32(par)
            s = sub0 + i
            pltpu.make_async_copy(idx_hbm.at[s], buf, sem_idx).wait()

            @pl.when(i < jnp.int32(nsub - 1))
            def _prefetch():
                nxt = buf1 if par == 0 else buf0
                pltpu.async_copy(idx_hbm.at[s + jnp.int32(1)], nxt, sem_idx)

            def widx_body(row, _):
                ro = row * jnp.int32(_LANES)
                for r in range(rounds):
                    for g in range(ng):
                        v = buf[jnp.int32(r), row,
                                pl.ds(jnp.int32(g * 16), 16)]
                        widx_vs[r][pl.ds(ro + jnp.int32(g * 16), 16)] = (
                            lax.shift_right_logical(v, jnp.int32(5)))
                return 0

            lax.fori_loop(jnp.int32(0), jnp.int32(_SR), widx_body, 0)

            cps = [pltpu.async_copy(spm.at[widx_vs[r]], got_vs[r], sem_g)
                   for r in range(rounds)]
            for c in cps:
                c.wait()

            def and_body(row, _):
                ro = row * jnp.int32(_LANES)
                for g in range(ng):
                    o = ro + jnp.int32(g * 16)
                    acc = None
                    for r in range(rounds):
                        sh = buf[jnp.int32(r), row,
                                 pl.ds(jnp.int32(g * 16), 16)] & jnp.int32(31)
                        t = lax.shift_right_logical(got_vs[r][pl.ds(o, 16)],
                                                    sh)
                        acc = t if acc is None else acc & t
                    out_v[pl.ds(o, 16)] = (acc & jnp.int32(1)) ^ jnp.int32(1)
                return 0

            lax.fori_loop(jnp.int32(0), jnp.int32(_SR), and_body, 0)
            pltpu.sync_copy(out_v, out_hbm.at[pl.ds(s * jnp.int32(sub), sub)])
        return 0

    lax.fori_loop(jnp.int32(0), jnp.int32(nsub // 2), outer, 0)


def kernel(negative_batch, bit_array, mersenne, rounds):
    batch, num_neg, _ = negative_batch.shape
    b = batch * num_neg
    size = bit_array.shape[0]
    try:
        r_static = int(rounds)
    except Exception:
        r_static = int(math.ceil(size / 1_000_000 * math.log(2)))

    nrow = b // _LANES
    t3 = (negative_batch.astype(jnp.int32)
          .reshape(b, 3).transpose(1, 0).reshape(3, nrow, _LANES))

    blk = _BW * _LANES
    w_pad = -(-(size // 32 + 1) // blk) * blk
    size_pad = 32 * w_pad
    bits8 = (jnp.pad(bit_array, (0, size_pad - size)).astype(jnp.int8)
             .reshape(size_pad // 4096, 4096))
    wts = _pack_weights()
    wspec = pl.BlockSpec(
        (4096, _LANES), lambda i: (jnp.int32(0), jnp.int32(0)))
    packed = pl.pallas_call(
        _pack_body,
        grid=(size_pad // 4096 // _BW,),
        in_specs=[pl.BlockSpec(
            (_BW, 4096), lambda i: (i, jnp.int32(0))),
            wspec, wspec, wspec, wspec],
        out_specs=pl.BlockSpec(
            (_BW, _LANES), lambda i: (i, jnp.int32(0))),
        out_shape=jax.ShapeDtypeStruct((w_pad // _LANES, _LANES), jnp.int32),
    )(bits8, wts[0], wts[1], wts[2], wts[3])
    table = packed.reshape(w_pad)

    bc = b // _NCHUNK
    nrow_c = nrow // _NCHUNK
    chunk = bc // _NW

    mesh = plsc.VectorSubcoreMesh(
        core_axis_name="c", subcore_axis_name="s",
        num_cores=_NC, num_subcores=_NS)
    sc_gather = pl.kernel(
        functools.partial(_gather_body, rounds=r_static, chunk=chunk,
                          sub=_SUB),
        out_type=jax.ShapeDtypeStruct((bc,), jnp.int32),
        mesh=mesh,
        scratch_types=(
            [pltpu.VMEM((_SUB,), jnp.int32) for _ in range(2 * r_static)]
            + [pltpu.VMEM((r_static, _SR, _LANES), jnp.int32),
               pltpu.VMEM((r_static, _SR, _LANES), jnp.int32),
               pltpu.VMEM((_SUB,), jnp.int32),
               pltpu.SemaphoreType.DMA, pltpu.SemaphoreType.DMA,
               pltpu.VMEM_SHARED((w_pad,), jnp.int32)]
        ),
    )

    outs = []
    for c in range(_NCHUNK):
        tc = lax.slice_in_dim(t3, c * nrow_c, (c + 1) * nrow_c, axis=1)
        idx = pl.pallas_call(
            functools.partial(_hash_body, rounds=r_static, size=size),
            grid=(nrow_c // _BR,),
            in_specs=[
                pl.BlockSpec(
                    (3, _BR, _LANES),
                    lambda i: (jnp.int32(0), i, jnp.int32(0))),
                pl.BlockSpec(
                    (_BW, _LANES),
                    lambda i: (jnp.int32(0), jnp.int32(0))),
            ],
            out_specs=pl.BlockSpec(
                (_BR // _SR, r_static, _SR, _LANES),
                lambda i: (i, jnp.int32(0), jnp.int32(0), jnp.int32(0))),
            out_shape=jax.ShapeDtypeStruct(
                (nrow_c // _SR, r_static, _SR, _LANES), jnp.int32),
        )(tc, packed)
        outs.append(sc_gather(idx, table))

    out = jnp.concatenate(outs)
    return out.reshape(batch, num_neg).astype(bool)


# speedup vs baseline: 2.1131x; 1.0186x over previous
import functools
import math

import jax
import jax.numpy as jnp
from jax import lax
from jax.experimental import pallas as pl
from jax.experimental.pallas import tpu as pltpu
from jax.experimental.pallas import tpu_sc as plsc

_C1 = 2146121005
_C2 = 2221713035
_MERSENNE = (2**17 - 1, 2**19 - 1, 2**31 - 1)
_LANES = 128
_NC, _NS = 2, 16
_NW = _NC * _NS
_BR = 32
_BW = 64
_SUB = 2048
_SR = _SUB // _LANES
_NCHUNK = 8


def _u(v):
    return jnp.uint32(v)


def _asr(x_u32, n):
    xi = lax.bitcast_convert_type(x_u32, jnp.int32)
    return lax.bitcast_convert_type(
        lax.shift_right_arithmetic(xi, jnp.int32(n)), jnp.uint32)


def _wide_mul_const(a, c):
    m16 = _u(0xFFFF)
    a0 = a & m16
    a1 = a >> _u(16)
    b0 = _u(c & 0xFFFF)
    b1 = _u((c >> 16) & 0xFFFF)
    w0 = a0 * b0
    t = a1 * b0 + (w0 >> _u(16))
    t2 = a0 * b1 + (t & m16)
    lo = (t2 << _u(16)) | (w0 & m16)
    hi = a1 * b1 + (t >> _u(16)) + (t2 >> _u(16))
    return hi, lo


def _xs(hi, lo, n):
    s_hi = _asr(hi, n)
    s_lo = (hi << _u(32 - n)) | (lo >> _u(n))
    return hi ^ s_hi, lo ^ s_lo


def _mc(hi, lo, c):
    ph, plo = _wide_mul_const(lo, c)
    return hi * _u(c) + ph, plo


def _mod_size(hi, lo, size):
    r32 = (1 << 32) % size
    s64 = (1 << 64) % size
    magic = (1 << 55) // size
    h, l = hi, lo
    for _ in range(3):
        ph, plo = _wide_mul_const(h, r32)
        l2 = plo + l
        carry = jnp.where(l2 < plo, _u(1), _u(0))
        h, l = ph + carry, l2
    s = h * _u(r32)
    v = s + l
    add1 = jnp.where(v < s, _u(r32), _u(0))
    v1 = v + add1
    add2 = jnp.where(v1 < add1, _u(r32), _u(0))
    v = v1 + add2
    qh, _ = _wide_mul_const(v, magic)
    q = qh >> _u(23)
    r = v - q * _u(size)
    r = jnp.where(r >= _u(size), r - _u(size), r)
    neg = lax.bitcast_convert_type(hi, jnp.int32) < jnp.int32(0)
    r_neg = r + jnp.where(r < _u(s64), _u(size), _u(0)) - _u(s64)
    r = jnp.where(neg, r_neg, r)
    return lax.bitcast_convert_type(r, jnp.int32)


def _pack_body(b_ref, w0, w1, w2, w3, out_ref):
    x = b_ref[...]
    dn = (((1,), (0,)), ((), ()))
    m = jnp.int32(0xFF)
    accs = []
    for w in (w0, w1, w2, w3):
        accs.append(lax.dot_general(
            x, w[...], dn, preferred_element_type=jnp.int32) & m)
    out_ref[...] = (accs[0] | (accs[1] << jnp.int32(8))
                    | (accs[2] << jnp.int32(16))
                    | (accs[3] << jnp.int32(24)))


def _pack_weights():
    import numpy as np
    w = np.zeros((4, 4096, 128), np.int16)
    for l in range(128):
        q, rem = divmod(l, 4)
        for h in range(4):
            for k in range(8):
                lane = 32 * rem + 8 * h + k
                w[h, 128 * q + lane, l] = 1 << k
    return jnp.asarray(w.astype(np.int8))


def _hash_body(t_ref, dep_ref, idx_ref, *, rounds, size):
    del dep_ref
    t0 = t_ref[0].astype(jnp.uint32)
    t1 = t_ref[1].astype(jnp.uint32)
    t2 = t_ref[2].astype(jnp.uint32)
    hi = jnp.zeros(t0.shape, jnp.uint32)
    lo = jnp.zeros(t0.shape, jnp.uint32)
    for m, tk in zip(_MERSENNE, (t0, t1, t2)):
        ph, plo = _wide_mul_const(tk, m)
        l2 = lo + plo
        carry = jnp.where(l2 < plo, _u(1), _u(0))
        hi, lo = hi + ph + carry, l2
    nq = _BR // _SR
    for r in range(rounds):
        hi, lo = _xs(hi, lo, 16)
        hi, lo = _mc(hi, lo, _C1)
        hi, lo = _xs(hi, lo, 15)
        hi, lo = _mc(hi, lo, _C2)
        hi, lo = _xs(hi, lo, 16)
        val = _mod_size(hi, lo, size)
        for q in range(nq):
            idx_ref[q, r] = val[q * _SR:(q + 1) * _SR]


def _gather_body(idx_hbm, table_hbm, out_hbm, *refs, rounds, chunk, sub):
    widx_vs = refs[:rounds]
    got_vs = refs[rounds:2 * rounds]
    buf0 = refs[2 * rounds]
    buf1 = refs[2 * rounds + 1]
    out_v = refs[2 * rounds + 2]
    sem_idx = refs[2 * rounds + 3]
    sem_g = refs[2 * rounds + 4]
    spm = refs[2 * rounds + 5]

    sid = lax.axis_index("s")

    @pl.when(sid == jnp.int32(0))
    def _stage():
        pltpu.sync_copy(table_hbm, spm)

    plsc.subcore_barrier()

    wid = sid * jnp.int32(_NC) + lax.axis_index("c")
    nsub = chunk // sub
    sub0 = wid * jnp.int32(nsub)
    ng = _LANES // 16

    pltpu.async_copy(idx_hbm.at[sub0], buf0, sem_idx)

    def outer(i2, _):
        for par, buf in ((0, buf0), (1, buf1)):
            i = i2 * jnp.int32(2) + jnp.int32(par)
            s = sub0 + i
            pltpu.make_async_copy(idx_hbm.at[s], buf, sem_idx).wait()

            @pl.when(i < jnp.int32(nsub - 1))
            def _prefetch():
                nxt = buf1 if par == 0 else buf0
                pltpu.async_copy(idx_hbm.at[s + jnp.int32(1)], nxt, sem_idx)

            def widx_body(row, _):
                ro = row * jnp.int32(_LANES)
                for r in range(rounds):
                    for g in range(ng):
                        v = buf[jnp.int32(r), row,
                                pl.ds(jnp.int32(g * 16), 16)]
                        widx_vs[r][pl.ds(ro + jnp.int32(g * 16), 16)] = (
                            lax.shift_right_logical(v, jnp.int32(5)))
                return 0

            lax.fori_loop(jnp.int32(0), jnp.int32(_SR), widx_body, 0)

            cps = [pltpu.async_copy(spm.at[widx_vs[r]], got_vs[r], sem_g)
                   for r in range(rounds)]
            for c in cps:
                c.wait()

            def and_body(row, _):
                ro = row * jnp.int32(_LANES)
                for g in range(ng):
                    o = ro + jnp.int32(g * 16)
                    acc = None
                    for r in range(rounds):
                        sh = buf[jnp.int32(r), row,
                                 pl.ds(jnp.int32(g * 16), 16)] & jnp.int32(31)
                        t = lax.shift_right_logical(got_vs[r][pl.ds(o, 16)],
                                                    sh)
                        acc = t if acc is None else acc & t
                    out_v[pl.ds(o, 16)] = (acc & jnp.int32(1)) ^ jnp.int32(1)
                return 0

            lax.fori_loop(jnp.int32(0), jnp.int32(_SR), and_body, 0)
            pltpu.sync_copy(out_v, out_hbm.at[pl.ds(s * jnp.int32(sub), sub)])
        return 0

    lax.fori_loop(jnp.int32(0), jnp.int32(nsub // 2), outer, 0)


def kernel(negative_batch, bit_array, mersenne, rounds):
    batch, num_neg, _ = negative_batch.shape
    b = batch * num_neg
    size = bit_array.shape[0]
    try:
        r_static = int(rounds)
    except Exception:
        r_static = int(math.ceil(size / 1_000_000 * math.log(2)))

    nrow = b // _LANES
    t3 = (negative_batch.astype(jnp.int32)
          .reshape(b, 3).transpose(1, 0).reshape(3, nrow, _LANES))

    blk = _BW * _LANES
    w_pad = -(-(size // 32 + 1) // blk) * blk
    size_pad = 32 * w_pad
    bits8 = (jnp.pad(bit_array, (0, size_pad - size)).astype(jnp.int8)
             .reshape(size_pad // 4096, 4096))
    wts = _pack_weights()
    wspec = pl.BlockSpec(
        (4096, _LANES), lambda i: (jnp.int32(0), jnp.int32(0)))
    packed = pl.pallas_call(
        _pack_body,
        grid=(size_pad // 4096 // _BW,),
        in_specs=[pl.BlockSpec(
            (_BW, 4096), lambda i: (i, jnp.int32(0))),
            wspec, wspec, wspec, wspec],
        out_specs=pl.BlockSpec(
            (_BW, _LANES), lambda i: (i, jnp.int32(0))),
        out_shape=jax.ShapeDtypeStruct((w_pad // _LANES, _LANES), jnp.int32),
    )(bits8, wts[0], wts[1], wts[2], wts[3])
    table = packed.reshape(w_pad)

    bc = b // _NCHUNK
    nrow_c = nrow // _NCHUNK
    chunk = bc // _NW

    mesh = plsc.VectorSubcoreMesh(
        core_axis_name="c", subcore_axis_name="s",
        num_cores=_NC, num_subcores=_NS)
    sc_gather = pl.kernel(
        functools.partial(_gather_body, rounds=r_static, chunk=chunk,
                          sub=_SUB),
        out_type=jax.ShapeDtypeStruct((bc,), jnp.int32),
        mesh=mesh,
        scratch_types=(
            [pltpu.VMEM((_SUB,), jnp.int32) for _ in range(2 * r_static)]
            + [pltpu.VMEM((r_static, _SR, _LANES), jnp.int32),
               pltpu.VMEM((r_static, _SR, _LANES), jnp.int32),
               pltpu.VMEM((_SUB,), jnp.int32),
               pltpu.SemaphoreType.DMA, pltpu.SemaphoreType.DMA,
               pltpu.VMEM_SHARED((w_pad,), jnp.int32)]
        ),
    )

    outs = []
    for c in range(_NCHUNK):
        tc = lax.slice_in_dim(t3, c * nrow_c, (c + 1) * nrow_c, axis=1)
        idx = pl.pallas_call(
            functools.partial(_hash_body, rounds=r_static, size=size),
            grid=(nrow_c // _BR,),
            in_specs=[
                pl.BlockSpec(
                    (3, _BR, _LANES),
                    lambda i: (jnp.int32(0), i, jnp.int32(0))),
                pl.BlockSpec(
                    (_BW, _LANES),
                    lambda i: (jnp.int32(0), jnp.int32(0))),
            ],
            out_specs=pl.BlockSpec(
                (_BR // _SR, r_static, _SR, _LANES),
                lambda i: (i, jnp.int32(0), jnp.int32(0), jnp.int32(0))),
            out_shape=jax.ShapeDtypeStruct(
                (nrow_c // _SR, r_static, _SR, _LANES), jnp.int32),
        )(tc, packed)
        outs.append(sc_gather(idx, table))

    out = jnp.concatenate(outs)
    return out.reshape(batch, num_neg).astype(bool)


# speedup vs baseline: 2.1655x; 1.0248x over previous
import functools
import math

import jax
import jax.numpy as jnp
from jax import lax
from jax.experimental import pallas as pl
from jax.experimental.pallas import tpu as pltpu
from jax.experimental.pallas import tpu_sc as plsc

_C1 = 2146121005
_C2 = 2221713035
_MERSENNE = (2**17 - 1, 2**19 - 1, 2**31 - 1)
_LANES = 128
_NC, _NS = 2, 16
_NW = _NC * _NS
_BR = 64
_BW = 64
_SUB = 2048
_SR = _SUB // _LANES
_NCHUNK = 8


def _u(v):
    return jnp.uint32(v)


def _asr(x_u32, n):
    xi = lax.bitcast_convert_type(x_u32, jnp.int32)
    return lax.bitcast_convert_type(
        lax.shift_right_arithmetic(xi, jnp.int32(n)), jnp.uint32)


def _wide_mul_const(a, c):
    m16 = _u(0xFFFF)
    a0 = a & m16
    a1 = a >> _u(16)
    b0 = _u(c & 0xFFFF)
    b1 = _u((c >> 16) & 0xFFFF)
    w0 = a0 * b0
    t = a1 * b0 + (w0 >> _u(16))
    t2 = a0 * b1 + (t & m16)
    lo = (t2 << _u(16)) | (w0 & m16)
    hi = a1 * b1 + (t >> _u(16)) + (t2 >> _u(16))
    return hi, lo


def _xs(hi, lo, n):
    s_hi = _asr(hi, n)
    s_lo = (hi << _u(32 - n)) | (lo >> _u(n))
    return hi ^ s_hi, lo ^ s_lo


def _mc(hi, lo, c):
    ph, plo = _wide_mul_const(lo, c)
    return hi * _u(c) + ph, plo


def _mod_size(hi, lo, size):
    r32 = (1 << 32) % size
    s64 = (1 << 64) % size
    magic = (1 << 55) // size
    h, l = hi, lo
    for _ in range(3):
        ph, plo = _wide_mul_const(h, r32)
        l2 = plo + l
        carry = jnp.where(l2 < plo, _u(1), _u(0))
        h, l = ph + carry, l2
    s = h * _u(r32)
    v = s + l
    add1 = jnp.where(v < s, _u(r32), _u(0))
    v1 = v + add1
    add2 = jnp.where(v1 < add1, _u(r32), _u(0))
    v = v1 + add2
    qh, _ = _wide_mul_const(v, magic)
    q = qh >> _u(23)
    r = v - q * _u(size)
    r = jnp.where(r >= _u(size), r - _u(size), r)
    neg = lax.bitcast_convert_type(hi, jnp.int32) < jnp.int32(0)
    r_neg = r + jnp.where(r < _u(s64), _u(size), _u(0)) - _u(s64)
    r = jnp.where(neg, r_neg, r)
    return lax.bitcast_convert_type(r, jnp.int32)


def _pack_body(b_ref, w0, w1, w2, w3, out_ref):
    x = b_ref[...]
    dn = (((1,), (0,)), ((), ()))
    m = jnp.int32(0xFF)
    accs = []
    for w in (w0, w1, w2, w3):
        accs.append(lax.dot_general(
            x, w[...], dn, preferred_element_type=jnp.int32) & m)
    out_ref[...] = (accs[0] | (accs[1] << jnp.int32(8))
                    | (accs[2] << jnp.int32(16))
                    | (accs[3] << jnp.int32(24)))


def _pack_weights():
    import numpy as np
    w = np.zeros((4, 4096, 128), np.int16)
    for l in range(128):
        q, rem = divmod(l, 4)
        for h in range(4):
            for k in range(8):
                lane = 32 * rem + 8 * h + k
                w[h, 128 * q + lane, l] = 1 << k
    return jnp.asarray(w.astype(np.int8))


def _hash_body(t_ref, dep_ref, idx_ref, *, rounds, size):
    del dep_ref
    t0 = t_ref[0].astype(jnp.uint32)
    t1 = t_ref[1].astype(jnp.uint32)
    t2 = t_ref[2].astype(jnp.uint32)
    hi = jnp.zeros(t0.shape, jnp.uint32)
    lo = jnp.zeros(t0.shape, jnp.uint32)
    for m, tk in zip(_MERSENNE, (t0, t1, t2)):
        ph, plo = _wide_mul_const(tk, m)
        l2 = lo + plo
        carry = jnp.where(l2 < plo, _u(1), _u(0))
        hi, lo = hi + ph + carry, l2
    nq = _BR // _SR
    for r in range(rounds):
        hi, lo = _xs(hi, lo, 16)
        hi, lo = _mc(hi, lo, _C1)
        hi, lo = _xs(hi, lo, 15)
        hi, lo = _mc(hi, lo, _C2)
        hi, lo = _xs(hi, lo, 16)
        val = _mod_size(hi, lo, size)
        for q in range(nq):
            idx_ref[q, r] = val[q * _SR:(q + 1) * _SR]


def _gather_body(idx_hbm, table_hbm, out_hbm, *refs, rounds, chunk, sub):
    widx_vs = refs[:rounds]
    got_vs = refs[rounds:2 * rounds]
    buf0 = refs[2 * rounds]
    buf1 = refs[2 * rounds + 1]
    out_v = refs[2 * rounds + 2]
    sem_idx = refs[2 * rounds + 3]
    sem_g = refs[2 * rounds + 4]
    spm = refs[2 * rounds + 5]

    sid = lax.axis_index("s")

    @pl.when(sid == jnp.int32(0))
    def _stage():
        pltpu.sync_copy(table_hbm, spm)

    plsc.subcore_barrier()

    wid = sid * jnp.int32(_NC) + lax.axis_index("c")
    nsub = chunk // sub
    sub0 = wid * jnp.int32(nsub)
    ng = _LANES // 16

    pltpu.async_copy(idx_hbm.at[sub0], buf0, sem_idx)

    def outer(i2, _):
        for par, buf in ((0, buf0), (1, buf1)):
            i = i2 * jnp.int32(2) + jnp.int32(par)
            s = sub0 + i
            pltpu.make_async_copy(idx_hbm.at[s], buf, sem_idx).wait()

            @pl.when(i < jnp.int32(nsub - 1))
            def _prefetch():
                nxt = buf1 if par == 0 else buf0
                pltpu.async_copy(idx_hbm.at[s + jnp.int32(1)], nxt, sem_idx)

            def widx_body(row, _):
                ro = row * jnp.int32(_LANES)
                for r in range(rounds):
                    for g in range(ng):
                        v = buf[jnp.int32(r), row,
                                pl.ds(jnp.int32(g * 16), 16)]
                        widx_vs[r][pl.ds(ro + jnp.int32(g * 16), 16)] = (
                            lax.shift_right_logical(v, jnp.int32(5)))
                return 0

            lax.fori_loop(jnp.int32(0), jnp.int32(_SR), widx_body, 0)

            cps = [pltpu.async_copy(spm.at[widx_vs[r]], got_vs[r], sem_g)
                   for r in range(rounds)]
            for c in cps:
                c.wait()

            def and_body(row, _):
                ro = row * jnp.int32(_LANES)
                for g in range(ng):
                    o = ro + jnp.int32(g * 16)
                    acc = None
                    for r in range(rounds):
                        sh = buf[jnp.int32(r), row,
                                 pl.ds(jnp.int32(g * 16), 16)] & jnp.int32(31)
                        t = lax.shift_right_logical(got_vs[r][pl.ds(o, 16)],
                                                    sh)
                        acc = t if acc is None else acc & t
                    out_v[pl.ds(o, 16)] = (acc & jnp.int32(1)) ^ jnp.int32(1)
                return 0

            lax.fori_loop(jnp.int32(0), jnp.int32(_SR), and_body, 0)
            pltpu.sync_copy(out_v, out_hbm.at[pl.ds(s * jnp.int32(sub), sub)])
        return 0

    lax.fori_loop(jnp.int32(0), jnp.int32(nsub // 2), outer, 0)


def kernel(negative_batch, bit_array, mersenne, rounds):
    batch, num_neg, _ = negative_batch.shape
    b = batch * num_neg
    size = bit_array.shape[0]
    try:
        r_static = int(rounds)
    except Exception:
        r_static = int(math.ceil(size / 1_000_000 * math.log(2)))

    nrow = b // _LANES
    t3 = (negative_batch.astype(jnp.int32)
          .reshape(b, 3).transpose(1, 0).reshape(3, nrow, _LANES))

    blk = _BW * _LANES
    w_pad = -(-(size // 32 + 1) // blk) * blk
    size_pad = 32 * w_pad
    bits8 = (jnp.pad(bit_array, (0, size_pad - size)).astype(jnp.int8)
             .reshape(size_pad // 4096, 4096))
    wts = _pack_weights()
    wspec = pl.BlockSpec(
        (4096, _LANES), lambda i: (jnp.int32(0), jnp.int32(0)))
    packed = pl.pallas_call(
        _pack_body,
        grid=(size_pad // 4096 // _BW,),
        in_specs=[pl.BlockSpec(
            (_BW, 4096), lambda i: (i, jnp.int32(0))),
            wspec, wspec, wspec, wspec],
        out_specs=pl.BlockSpec(
            (_BW, _LANES), lambda i: (i, jnp.int32(0))),
        out_shape=jax.ShapeDtypeStruct((w_pad // _LANES, _LANES), jnp.int32),
    )(bits8, wts[0], wts[1], wts[2], wts[3])
    table = packed.reshape(w_pad)

    bc = b // _NCHUNK
    nrow_c = nrow // _NCHUNK
    chunk = bc // _NW

    mesh = plsc.VectorSubcoreMesh(
        core_axis_name="c", subcore_axis_name="s",
        num_cores=_NC, num_subcores=_NS)
    sc_gather = pl.kernel(
        functools.partial(_gather_body, rounds=r_static, chunk=chunk,
                          sub=_SUB),
        out_type=jax.ShapeDtypeStruct((bc,), jnp.int32),
        mesh=mesh,
        scratch_types=(
            [pltpu.VMEM((_SUB,), jnp.int32) for _ in range(2 * r_static)]
            + [pltpu.VMEM((r_static, _SR, _LANES), jnp.int32),
               pltpu.VMEM((r_static, _SR, _LANES), jnp.int32),
               pltpu.VMEM((_SUB,), jnp.int32),
               pltpu.SemaphoreType.DMA, pltpu.SemaphoreType.DMA,
               pltpu.VMEM_SHARED((w_pad,), jnp.int32)]
        ),
    )

    outs = []
    for c in range(_NCHUNK):
        tc = lax.slice_in_dim(t3, c * nrow_c, (c + 1) * nrow_c, axis=1)
        idx = pl.pallas_call(
            functools.partial(_hash_body, rounds=r_static, size=size),
            grid=(nrow_c // _BR,),
            in_specs=[
                pl.BlockSpec(
                    (3, _BR, _LANES),
                    lambda i: (jnp.int32(0), i, jnp.int32(0))),
                pl.BlockSpec(
                    (_BW, _LANES),
                    lambda i: (jnp.int32(0), jnp.int32(0))),
            ],
            out_specs=pl.BlockSpec(
                (_BR // _SR, r_static, _SR, _LANES),
                lambda i: (i, jnp.int32(0), jnp.int32(0), jnp.int32(0))),
            out_shape=jax.ShapeDtypeStruct(
                (nrow_c // _SR, r_static, _SR, _LANES), jnp.int32),
        )(tc, packed)
        outs.append(sc_gather(idx, table))

    out = jnp.concatenate(outs)
    return out.reshape(batch, num_neg).astype(bool)
